# Initial kernel scaffold; baseline (speedup 1.0000x reference)
import jax
import jax.numpy as jnp
from jax.experimental import pallas as pl


def kernel(texts_tokenized, table):
    raise NotImplementedError("write your pallas kernel here")


# speedup vs baseline: 1.4593x; 1.4593x over previous
import functools

import jax
import jax.numpy as jnp
from jax import lax
from jax.experimental import pallas as pl
from jax.experimental.pallas import tpu as pltpu
from jax.experimental.pallas import tpu_sc as plsc

DEPTH = 32
NUM_TOKENS = 4096 * 200
NC = 2
NS = 16
NW = NC * NS
PER_W = NUM_TOKENS // NW
CHUNK = 1024
NCHUNK = PER_W // CHUNK

_mesh = plsc.VectorSubcoreMesh(core_axis_name="c", subcore_axis_name="s")


@functools.partial(
    pl.kernel,
    mesh=_mesh,
    compiler_params=pltpu.CompilerParams(use_tc_tiling_on_sc=False),
    out_type=jax.ShapeDtypeStruct((NUM_TOKENS, DEPTH), jnp.float32),
    scratch_types=[
        pltpu.VMEM((CHUNK,), jnp.int32),
        pltpu.VMEM((CHUNK, DEPTH), jnp.float32),
        pltpu.SemaphoreType.DMA,
    ],
)
def _embed_lookup(idx_hbm, table_hbm, out_hbm, idx_v, rows_v, sem):
    wid = lax.axis_index("s") * NC + lax.axis_index("c")
    base = wid * PER_W

    def body(i, carry):
        off = base + i * CHUNK
        pltpu.sync_copy(idx_hbm.at[pl.ds(off, CHUNK)], idx_v)
        pltpu.async_copy(table_hbm.at[idx_v], rows_v, sem).wait()
        pltpu.sync_copy(rows_v, out_hbm.at[pl.ds(off, CHUNK)])
        return carry

    lax.fori_loop(0, NCHUNK, body, 0)


def kernel(texts_tokenized, table):
    idx = texts_tokenized.reshape(-1).astype(jnp.int32)
    out = _embed_lookup(idx, table)
    return out.reshape(texts_tokenized.shape + (DEPTH,))


# speedup vs baseline: 1.5026x; 1.0297x over previous
import functools

import jax
import jax.numpy as jnp
from jax import lax
from jax.experimental import pallas as pl
from jax.experimental.pallas import tpu as pltpu
from jax.experimental.pallas import tpu_sc as plsc

DEPTH = 32
NUM_TOKENS = 4096 * 200
NC = 2
NS = 16
NW = NC * NS
PER_W = NUM_TOKENS // NW
CHUNK = 800
NCHUNK = PER_W // CHUNK

_mesh = plsc.VectorSubcoreMesh(core_axis_name="c", subcore_axis_name="s")


@functools.partial(
    pl.kernel,
    mesh=_mesh,
    compiler_params=pltpu.CompilerParams(use_tc_tiling_on_sc=False),
    out_type=jax.ShapeDtypeStruct((NUM_TOKENS, DEPTH), jnp.float32),
    scratch_types=[
        pltpu.VMEM((NCHUNK, CHUNK), jnp.int32),
        pltpu.VMEM((CHUNK, DEPTH), jnp.float32),
        pltpu.VMEM((CHUNK, DEPTH), jnp.float32),
        pltpu.SemaphoreType.DMA,
        pltpu.SemaphoreType.DMA,
        pltpu.SemaphoreType.DMA,
        pltpu.SemaphoreType.DMA,
    ],
)
def _embed_lookup(idx_hbm, table_hbm, out_hbm, idx_v, rows0, rows1,
                  sg0, sg1, so0, so1):
    wid = lax.axis_index("s") * NC + lax.axis_index("c")
    base = wid * PER_W
    pltpu.sync_copy(idx_hbm.at[wid], idx_v)

    rows = (rows0, rows1)
    sg = (sg0, sg1)
    so = (so0, so1)

    def g_copy(i, b):
        return pltpu.make_async_copy(table_hbm.at[idx_v.at[i]], rows[b], sg[b])

    def o_copy(i, b):
        return pltpu.make_async_copy(
            rows[b], out_hbm.at[pl.ds(base + i * CHUNK, CHUNK)], so[b])

    g_copy(0, 0).start()
    g_copy(1, 1).start()

    def step(i, b):
        g_copy(i, b).wait()
        o_copy(i, b).start()
        o_copy(i, b).wait()
        g_copy(i + 2, b).start()

    def body(k, carry):
        i = 2 * k
        step(i, 0)
        step(i + 1, 1)
        return carry

    lax.fori_loop(0, (NCHUNK - 2) // 2, body, 0)

    i = NCHUNK - 2
    g_copy(i, 0).wait()
    o_copy(i, 0).start()
    g_copy(i + 1, 1).wait()
    o_copy(i + 1, 1).start()
    o_copy(i, 0).wait()
    o_copy(i + 1, 1).wait()


def kernel(texts_tokenized, table):
    idx = texts_tokenized.reshape(NW, NCHUNK, CHUNK).astype(jnp.int32)
    out = _embed_lookup(idx, table)
    return out.reshape(texts_tokenized.shape + (DEPTH,))
